# baseline (speedup 1.0000x reference)
import functools

import jax
import jax.numpy as jnp
from jax import lax
from jax.experimental import pallas as pl
from jax.experimental.pallas import tpu as pltpu
from jax.experimental.pallas import tpu_sc as plsc

N = 1000000
C = 32
NUM_OUT = 262144
NUM_RANGES = 8
S = NUM_OUT // NUM_RANGES
PASSES = NUM_RANGES // 2
CHUNK = 2000
NCHUNK = N // CHUNK
VPC = CHUNK // 16
B = 128
ZB = 64
SPAN = S // 16
STAGE = CHUNK + 2 * B


def _body(feat_hbm, coords_hbm, out_hbm,
          sums_sh, cnts_sh,
          coords_v, ids_v, rel_v, rel_row, rows_v,
          ones_v, zer32_v, zer16_v, sums_o, cnts_o):
    c = lax.axis_index("c")
    s = lax.axis_index("s")
    iota = lax.iota(jnp.int32, 16)
    f_one = jnp.full((16,), 1.0, jnp.float32)
    f_zero = jnp.full((16,), 0.0, jnp.float32)
    i_zero = jnp.full((16,), 0, jnp.int32)
    i_trash = jnp.full((16,), S, jnp.int32)

    def init_ones(i, _):
        ones_v[i] = f_one
        return 0
    lax.fori_loop(0, B, init_ones, 0)

    def init_zeros(i, _):
        zer16_v[i] = f_zero
        zer32_v[i, pl.ds(0, 16)] = f_zero
        zer32_v[i, pl.ds(16, 16)] = f_zero
        return 0
    lax.fori_loop(0, ZB, init_zeros, 0)

    n_my = jnp.where(s < NCHUNK - 16 * (NCHUNK // 16), NCHUNK // 16 + 1,
                     NCHUNK // 16)

    for p in range(PASSES):
        rng = c * PASSES + p
        base = rng * S

        for blk in range(SPAN // ZB):
            r0 = s * SPAN + blk * ZB
            pltpu.sync_copy(zer32_v, sums_sh.at[pl.ds(r0, ZB)])
            pltpu.sync_copy(zer16_v, cnts_sh.at[pl.ds(r0, ZB)])
        plsc.subcore_barrier()

        def do_chunk(k, _):
            chunk_id = s + k * 16
            point0 = chunk_id * CHUNK
            pltpu.sync_copy(coords_hbm.at[pl.ds(point0 * 3, CHUNK * 3)],
                            coords_v)

            def scan16(i, off):
                i3 = i * 48 + iota * 3
                x = plsc.load_gather(coords_v, [i3])
                y = plsc.load_gather(coords_v, [i3 + 1])
                z = plsc.load_gather(coords_v, [i3 + 2])
                seg = ((x >> 1) << 12) | ((y >> 1) << 6) | (z >> 1)
                m = (seg >> 15) == rng
                rel = seg & (S - 1)
                pid = point0 + i * 16 + iota
                plsc.store_compressed(ids_v.at[pl.ds(off, 16)], pid, mask=m)
                plsc.store_compressed(rel_v.at[pl.ds(off, 16)], rel, mask=m)
                return off + jnp.sum(m.astype(jnp.int32))

            m_cnt = lax.fori_loop(0, VPC, scan16, 0)

            for j in range(B // 16):
                ids_v[pl.ds(m_cnt + j * 16, 16)] = i_zero
                rel_v[pl.ds(m_cnt + j * 16, 16)] = i_trash

            def do_batch(j, _):
                o = j * B
                for t in range(B // 16):
                    rel_row[0, pl.ds(t * 16, 16)] = rel_v[pl.ds(o + t * 16, 16)]
                pltpu.sync_copy(feat_hbm.at[ids_v.at[pl.ds(o, B)]], rows_v)
                pltpu.sync_copy(rows_v, sums_sh.at[rel_row.at[0]], add=True)
                pltpu.sync_copy(ones_v, cnts_sh.at[rel_row.at[0]], add=True)
                return 0

            lax.fori_loop(0, (m_cnt + B - 1) // B, do_batch, 0)
            return 0

        lax.fori_loop(0, n_my, do_chunk, 0)
        plsc.subcore_barrier()

        for blk in range(SPAN // B):
            r0 = s * SPAN + blk * B
            pltpu.sync_copy(sums_sh.at[pl.ds(r0, B)], sums_o)
            pltpu.sync_copy(cnts_sh.at[pl.ds(r0, B)], cnts_o)

            def divrow(rr, _):
                cm = jnp.maximum(cnts_o[rr], 1.0)
                sums_o[rr, pl.ds(0, 16)] = sums_o[rr, pl.ds(0, 16)] / cm
                sums_o[rr, pl.ds(16, 16)] = sums_o[rr, pl.ds(16, 16)] / cm
                return 0

            lax.fori_loop(0, B, divrow, 0)
            pltpu.sync_copy(sums_o, out_hbm.at[pl.ds(base + r0, B)])
        plsc.subcore_barrier()


@jax.jit
def _pooled(features, coords):
    mesh = plsc.VectorSubcoreMesh(core_axis_name="c", subcore_axis_name="s")
    f = pl.kernel(
        _body,
        out_type=jax.ShapeDtypeStruct((NUM_OUT, C), jnp.float32),
        mesh=mesh,
        compiler_params=pltpu.CompilerParams(needs_layout_passes=False,
                                             use_tc_tiling_on_sc=False),
        scratch_types=[
            pltpu.VMEM_SHARED((S + 1, C), jnp.float32),
            pltpu.VMEM_SHARED((S + 1, 16), jnp.float32),
            pltpu.VMEM((CHUNK * 3,), jnp.int32),
            pltpu.VMEM((STAGE,), jnp.int32),
            pltpu.VMEM((STAGE,), jnp.int32),
            pltpu.VMEM((1, B), jnp.int32),
            pltpu.VMEM((B, C), jnp.float32),
            pltpu.VMEM((B, 16), jnp.float32),
            pltpu.VMEM((ZB, C), jnp.float32),
            pltpu.VMEM((ZB, 16), jnp.float32),
            pltpu.VMEM((B, C), jnp.float32),
            pltpu.VMEM((B, 16), jnp.float32),
        ],
    )
    return f(features, coords)


def kernel(features, coords):
    return _pooled(features, coords.reshape(-1))
